# Initial kernel scaffold; baseline (speedup 1.0000x reference)
import jax
import jax.numpy as jnp
from jax.experimental import pallas as pl


def kernel(z, embedding):
    raise NotImplementedError("write your pallas kernel here")


# speedup vs baseline: 1.3178x; 1.3178x over previous
import functools

import jax
import jax.numpy as jnp
from jax import lax
from jax.experimental import pallas as pl
from jax.experimental.pallas import tpu as pltpu
from jax.experimental.pallas import tpu_sc as plsc

K = 8192
D = 32
B = 16
HW = 1024
KC = 2048


def _argmin_body(z_ref, emb_ref, idx_ref, bv_ref, bi_ref, hv_ref, hi_ref):
    c = pl.program_id(1)
    zb = z_ref[0]
    z2 = jnp.sum(zb * zb, axis=0, keepdims=True)
    e_c = emb_ref[...]
    e2 = jnp.sum(e_c * e_c, axis=1, keepdims=True)
    m = jax.lax.dot_general(
        e_c, zb, (((1,), (0,)), ((), ())),
        preferred_element_type=jnp.float32)
    scores = (z2 - 2.0 * m) + e2
    cmin = jnp.min(scores, axis=0, keepdims=True)
    rows = lax.broadcasted_iota(jnp.int32, (KC, HW), 0)
    cam = jnp.min(jnp.where(scores == cmin, rows, KC),
                  axis=0, keepdims=True) + c * KC

    @pl.when(c == 0)
    def _():
        bv_ref[...] = cmin
        bi_ref[...] = cam

    @pl.when(c == 1)
    def _():
        upd = cmin < bv_ref[...]
        half_min = jnp.where(upd, cmin, bv_ref[...])
        bv_ref[...] = half_min.astype(jnp.bfloat16).astype(jnp.float32)
        bi_ref[...] = jnp.where(upd, cam, bi_ref[...])

    @pl.when(c == 2)
    def _():
        hv_ref[...] = cmin
        hi_ref[...] = cam

    @pl.when(c == 3)
    def _():
        upd2 = cmin < hv_ref[...]
        h2_min = jnp.where(upd2, cmin, hv_ref[...])
        h2_idx = jnp.where(upd2, cam, hi_ref[...])
        upd = h2_min < bv_ref[...]
        bi_ref[...] = jnp.where(upd, h2_idx, bi_ref[...])

    @pl.when(c == pl.num_programs(1) - 1)
    def _():
        idx_ref[0] = bi_ref[...]


def _compute_indices(z3, embedding):
    return pl.pallas_call(
        _argmin_body,
        grid=(B, K // KC),
        in_specs=[
            pl.BlockSpec((1, D, HW), lambda b, c: (b, 0, 0)),
            pl.BlockSpec((KC, D), lambda b, c: (c, 0)),
        ],
        out_specs=pl.BlockSpec((1, 1, HW), lambda b, c: (b, 0, 0)),
        out_shape=jax.ShapeDtypeStruct((B, 1, HW), jnp.int32),
        scratch_shapes=[
            pltpu.VMEM((1, HW), jnp.float32),
            pltpu.VMEM((1, HW), jnp.int32),
            pltpu.VMEM((1, HW), jnp.float32),
            pltpu.VMEM((1, HW), jnp.int32),
        ],
    )(z3, embedding)


@functools.cache
def _make_gather():
    info = plsc.get_sparse_core_info()
    nc, ns = info.num_cores, info.num_subcores
    nw = nc * ns
    n = B * HW
    per_w = n // nw
    mesh = plsc.VectorSubcoreMesh(core_axis_name="c", subcore_axis_name="s")

    @functools.partial(
        pl.kernel, mesh=mesh,
        compiler_params=pltpu.CompilerParams(use_tc_tiling_on_sc=False),
        out_type=jax.ShapeDtypeStruct((n, D), jnp.float32),
        scratch_types=[
            pltpu.VMEM((per_w,), jnp.int32),
            pltpu.VMEM((per_w, D), jnp.float32),
            pltpu.SemaphoreType.DMA,
        ],
    )
    def gather_rows(emb_hbm, idx_hbm, out_hbm, idx_v, rows_v, sem):
        wid = lax.axis_index("s") * nc + lax.axis_index("c")
        base = wid * per_w
        pltpu.sync_copy(idx_hbm.at[pl.ds(base, per_w)], idx_v)
        pltpu.async_copy(emb_hbm.at[idx_v], rows_v, sem).wait()
        pltpu.sync_copy(rows_v, out_hbm.at[pl.ds(base, per_w)])

    return gather_rows


def kernel(z, embedding):
    z3 = z.reshape(B, D, HW)
    idx3 = _compute_indices(z3, embedding)
    idx_flat = idx3.reshape(B * HW)
    q_flat = _make_gather()(embedding, idx_flat)
    quantized = q_flat.reshape(B, 32, 32, D).transpose(0, 3, 1, 2)
    return quantized, idx3.reshape(B, HW)
